# baseline (speedup 1.0000x reference)
import functools

import jax
import jax.numpy as jnp
from jax import lax
from jax.experimental import pallas as pl
from jax.experimental.pallas import tpu as pltpu
from jax.experimental.pallas import tpu_sc as plsc

BATCH = 4096
HIST = 200
D = 64
V = 1000002
NC, NS = 2, 16
NW = NC * NS

_mesh = plsc.VectorSubcoreMesh(core_axis_name="c", subcore_axis_name="s")
_params = pltpu.CompilerParams(use_tc_tiling_on_sc=True, needs_layout_passes=False)

CB = 256
NBF = V // CB
TAIL = V - NBF * CB
TAIL_PAD = 40
K1_ITER = NBF // NW + 1
VPP = NBF * CB // 2 + TAIL_PAD


def _transpose_block(blk, out, rows):
    iota = lax.iota(jnp.int32, 16)
    dvecs = [16 * kk + iota for kk in range(4)]

    def row(r, c):
        for k in range(8):
            i = 2 * r + (1 if k >= 4 else 0)
            vals = plsc.load_gather(blk, [dvecs[k % 4], jnp.full((16,), i, jnp.int32)])
            out[r, pl.ds(16 * k, 16)] = vals
        return c

    lax.fori_loop(0, rows, row, 0)


@functools.partial(
    pl.kernel,
    out_type=jax.ShapeDtypeStruct((VPP, 128), jnp.float32),
    mesh=_mesh,
    scratch_types=[
        pltpu.VMEM((2, 64, CB), jnp.float32),
        pltpu.VMEM((2, CB // 2, 128), jnp.float32),
        pltpu.SemaphoreType.DMA,
        pltpu.SemaphoreType.DMA,
        pltpu.SemaphoreType.DMA,
        pltpu.SemaphoreType.DMA,
    ],
    compiler_params=_params,
)
def _k1(tT_hbm, tail_hbm, pair_hbm, blk_v, out_v, rs0, rs1, ws0, ws1):
    wid = lax.axis_index("s") * NC + lax.axis_index("c")
    rsem = (rs0, rs1)
    wsem = (ws0, ws1)
    nblk = (NBF - wid + NW - 1) // NW

    def col0_of(it):
        return pl.multiple_of((wid + NW * it) * CB, CB)

    def start_read(it, slot):
        pltpu.async_copy(tT_hbm.at[:, pl.ds(col0_of(it), CB)], blk_v.at[slot],
                         rsem[slot])

    def wait_read(slot):
        pltpu.make_async_copy(tT_hbm.at[:, pl.ds(0, CB)], blk_v.at[slot],
                              rsem[slot]).wait()

    def start_write(it, slot):
        pltpu.async_copy(
            out_v.at[slot],
            pair_hbm.at[pl.ds(pl.multiple_of(col0_of(it) // 2, CB // 2), CB // 2)],
            wsem[slot])

    def wait_write(slot):
        pltpu.make_async_copy(out_v.at[slot], pair_hbm.at[pl.ds(0, CB // 2)],
                              wsem[slot]).wait()

    @pl.when(nblk >= 1)
    def _():
        start_read(0, 0)
    @pl.when(nblk >= 2)
    def _():
        start_read(1, 1)

    def step2(it2, c):
        for slot in range(2):
            it = 2 * it2 + slot

            @pl.when(it < nblk)
            def _():
                wait_read(slot)
                @pl.when(it >= 2)
                def _():
                    wait_write(slot)
                _transpose_block(blk_v.at[slot], out_v.at[slot], CB // 2)
                start_write(it, slot)
                @pl.when(it + 2 < nblk)
                def _():
                    start_read(it + 2, slot)
        return c

    lax.fori_loop(0, (K1_ITER + 1) // 2, step2, 0)

    for slot in range(2):
        @pl.when(nblk > slot)
        def _():
            wait_write(slot)

    @pl.when(wid == 0)
    def _():
        pltpu.sync_copy(tail_hbm, out_v.at[0, pl.ds(0, TAIL_PAD)])
        pltpu.sync_copy(out_v.at[0, pl.ds(0, TAIL_PAD)],
                        pair_hbm.at[pl.ds(NBF * CB // 2, TAIL_PAD)])


BB = 256
GB = 1024
NGB = BATCH // GB
GROUPS = HIST * NGB
G_PER_W = GROUPS // NW


@functools.partial(
    pl.kernel,
    out_type=jax.ShapeDtypeStruct((HIST, D, BATCH), jnp.float32),
    mesh=_mesh,
    scratch_types=[
        pltpu.VMEM((2, 8, 128), jnp.int32),
        pltpu.VMEM((2, 8, 128), jnp.int32),
        pltpu.VMEM((2, GB), jnp.int32),
        pltpu.VMEM((2, BB, 128), jnp.float32),
        pltpu.VMEM((2, D, BB), jnp.float32),
        pltpu.SemaphoreType.DMA,
        pltpu.SemaphoreType.DMA,
        pltpu.SemaphoreType.DMA,
        pltpu.SemaphoreType.DMA,
        pltpu.SemaphoreType.DMA,
        pltpu.SemaphoreType.DMA,
    ],
    compiler_params=_params,
)
def _k2(idx3_hbm, pair_hbm, o_hbm, raw_v, gidx_v, par_v, rows_v, ot_v,
        is0, is1, gs0, gs1, ws0, ws1):
    wid = lax.axis_index("s") * NC + lax.axis_index("c")
    iota = lax.iota(jnp.int32, 16)
    isem = (is0, is1)
    gsem = (gs0, gs1)
    wsem = (ws0, ws1)

    def gid_of(i):
        return wid + NW * i

    def start_idx_read(i, gs):
        pltpu.async_copy(idx3_hbm.at[gid_of(i)], raw_v.at[gs], isem[gs])

    def wait_idx_read(gs):
        pltpu.make_async_copy(idx3_hbm.at[0], raw_v.at[gs], isem[gs]).wait()

    def prep(gs):
        for q in range(GB // 16):
            v = raw_v[gs, q // 8, pl.ds(16 * (q % 8), 16)]
            gidx_v[gs, q // 8, pl.ds(16 * (q % 8), 16)] = v >> 1
            par_v[gs, pl.ds(16 * q, 16)] = (v & 1) * 64

    def start_gather(gs, s4, rslot):
        for q in range(2):
            pltpu.async_copy(pair_hbm.at[gidx_v.at[gs, 2 * s4 + q]],
                             rows_v.at[rslot, pl.ds(128 * q, 128)], gsem[rslot])

    def wait_gather(rslot):
        for q in range(2):
            pltpu.make_async_copy(pair_hbm.at[gidx_v.at[0, 0]],
                                  rows_v.at[rslot, pl.ds(0, 128)],
                                  gsem[rslot]).wait()

    def select_transpose(gs, s4, rslot):
        p64s = [par_v[gs, pl.ds(BB * s4 + 16 * m, 16)] for m in range(BB // 16)]
        bvecs = [16 * m + iota for m in range(BB // 16)]

        def drow(d, c):
            for m in range(BB // 16):
                vals = plsc.load_gather(rows_v.at[rslot], [bvecs[m], p64s[m] + d])
                ot_v[rslot, d, pl.ds(16 * m, 16)] = vals
            return c

        lax.fori_loop(0, D, drow, 0)

    def start_out_write(i, s4, rslot):
        g = gid_of(i)
        t = g // NGB
        b0 = pl.multiple_of((g % NGB) * GB + BB * s4, BB)
        pltpu.async_copy(ot_v.at[rslot], o_hbm.at[t, :, pl.ds(b0, BB)],
                         wsem[rslot])

    def wait_out_write(rslot):
        pltpu.make_async_copy(ot_v.at[rslot], o_hbm.at[0, :, pl.ds(0, BB)],
                              wsem[rslot]).wait()

    def process_group(i, gs, first, last):
        wait_idx_read(gs)
        if not last:
            start_idx_read(i + 1, 1 - gs)
        prep(gs)
        start_gather(gs, 0, 0)
        start_gather(gs, 1, 1)
        for s4 in range(4):
            rslot = s4 % 2
            wait_gather(rslot)
            if s4 < 2:
                if not first:
                    wait_out_write(rslot)
                else:
                    pass
            else:
                wait_out_write(rslot)
            select_transpose(gs, s4, rslot)
            start_out_write(i, s4, rslot)
            if s4 + 2 < 4:
                start_gather(gs, s4 + 2, rslot)

    start_idx_read(0, 0)

    def step2(i2, c):
        i = 2 * i2

        @pl.when(i2 == 0)
        def _():
            process_group(i, 0, True, False)
        @pl.when(i2 > 0)
        def _():
            process_group(i, 0, False, False)
        process_group(i + 1, 1, False, False)
        return c

    lax.fori_loop(0, (G_PER_W - 1) // 2, step2, 0)
    process_group(G_PER_W - 1, 0, False, True)

    wait_out_write(0)
    wait_out_write(1)


def kernel(idx, layer_matrix):
    pair = _k1(layer_matrix.T, _tail_pairs(layer_matrix))
    idx3 = idx.T.astype(jnp.int32).reshape(GROUPS, 8, 128)
    o = _k2(idx3, pair)
    return o.transpose(2, 0, 1)


def _tail_pairs(layer_matrix):
    lm_tail = lax.slice(layer_matrix, (NBF * CB, 0), (V, D))
    return jnp.pad(lm_tail, ((0, 2 * TAIL_PAD - TAIL), (0, 0))).reshape(TAIL_PAD, 128)


# speedup vs baseline: 1.2477x; 1.2477x over previous
import functools

import jax
import jax.numpy as jnp
from jax import lax
from jax.experimental import pallas as pl
from jax.experimental.pallas import tpu as pltpu
from jax.experimental.pallas import tpu_sc as plsc

BATCH = 4096
HIST = 200
D = 64
V = 1000002
NC, NS = 2, 16
NW = NC * NS

_mesh = plsc.VectorSubcoreMesh(core_axis_name="c", subcore_axis_name="s")
_params = pltpu.CompilerParams(use_tc_tiling_on_sc=True, needs_layout_passes=False)

CB = 256
NBF = V // CB
TAIL = V - NBF * CB
TAIL_PAD = 40
K1_ITER = NBF // NW + 1
VPP = NBF * CB // 2 + TAIL_PAD


def _transpose_block(blk, out, rows):
    iota = lax.iota(jnp.int32, 16)
    dvecs = [16 * kk + iota for kk in range(4)]

    def rowgrp(g, c):
        r0 = g * 8
        for rr in range(8):
            i_even = jnp.full((16,), 2 * (r0 + rr), jnp.int32)
            i_odd = i_even + 1
            vals = [
                plsc.load_gather(blk, [dvecs[k % 4], i_odd if k >= 4 else i_even])
                for k in range(8)
            ]
            for k in range(8):
                out[r0 + rr, pl.ds(16 * k, 16)] = vals[k]
        return c

    lax.fori_loop(0, rows // 8, rowgrp, 0)


@functools.partial(
    pl.kernel,
    out_type=jax.ShapeDtypeStruct((VPP, 128), jnp.float32),
    mesh=_mesh,
    scratch_types=[
        pltpu.VMEM((2, 64, CB), jnp.float32),
        pltpu.VMEM((2, CB // 2, 128), jnp.float32),
        pltpu.SemaphoreType.DMA,
        pltpu.SemaphoreType.DMA,
        pltpu.SemaphoreType.DMA,
        pltpu.SemaphoreType.DMA,
    ],
    compiler_params=_params,
)
def _k1(tT_hbm, tail_hbm, pair_hbm, blk_v, out_v, rs0, rs1, ws0, ws1):
    wid = lax.axis_index("s") * NC + lax.axis_index("c")
    rsem = (rs0, rs1)
    wsem = (ws0, ws1)
    nblk = (NBF - wid + NW - 1) // NW

    def col0_of(it):
        return pl.multiple_of((wid + NW * it) * CB, CB)

    def start_read(it, slot):
        pltpu.async_copy(tT_hbm.at[:, pl.ds(col0_of(it), CB)], blk_v.at[slot],
                         rsem[slot])

    def wait_read(slot):
        pltpu.make_async_copy(tT_hbm.at[:, pl.ds(0, CB)], blk_v.at[slot],
                              rsem[slot]).wait()

    def start_write(it, slot):
        pltpu.async_copy(
            out_v.at[slot],
            pair_hbm.at[pl.ds(pl.multiple_of(col0_of(it) // 2, CB // 2), CB // 2)],
            wsem[slot])

    def wait_write(slot):
        pltpu.make_async_copy(out_v.at[slot], pair_hbm.at[pl.ds(0, CB // 2)],
                              wsem[slot]).wait()

    @pl.when(nblk >= 1)
    def _():
        start_read(0, 0)
    @pl.when(nblk >= 2)
    def _():
        start_read(1, 1)

    def step2(it2, c):
        for slot in range(2):
            it = 2 * it2 + slot

            @pl.when(it < nblk)
            def _():
                wait_read(slot)
                @pl.when(it >= 2)
                def _():
                    wait_write(slot)
                _transpose_block(blk_v.at[slot], out_v.at[slot], CB // 2)
                start_write(it, slot)
                @pl.when(it + 2 < nblk)
                def _():
                    start_read(it + 2, slot)
        return c

    lax.fori_loop(0, (K1_ITER + 1) // 2, step2, 0)

    for slot in range(2):
        @pl.when(nblk > slot)
        def _():
            wait_write(slot)

    @pl.when(wid == 0)
    def _():
        pltpu.sync_copy(tail_hbm, out_v.at[0, pl.ds(0, TAIL_PAD)])
        pltpu.sync_copy(out_v.at[0, pl.ds(0, TAIL_PAD)],
                        pair_hbm.at[pl.ds(NBF * CB // 2, TAIL_PAD)])


BB = 256
GB = 1024
NGB = BATCH // GB
GROUPS = HIST * NGB
G_PER_W = GROUPS // NW


@functools.partial(
    pl.kernel,
    out_type=jax.ShapeDtypeStruct((HIST, D, BATCH), jnp.float32),
    mesh=_mesh,
    scratch_types=[
        pltpu.VMEM((2, 8, 128), jnp.int32),
        pltpu.VMEM((2, 8, 128), jnp.int32),
        pltpu.VMEM((2, GB), jnp.int32),
        pltpu.VMEM((2, BB, 128), jnp.float32),
        pltpu.VMEM((2, D, BB), jnp.float32),
        pltpu.SemaphoreType.DMA,
        pltpu.SemaphoreType.DMA,
        pltpu.SemaphoreType.DMA,
        pltpu.SemaphoreType.DMA,
        pltpu.SemaphoreType.DMA,
        pltpu.SemaphoreType.DMA,
    ],
    compiler_params=_params,
)
def _k2(idx3_hbm, pair_hbm, o_hbm, raw_v, gidx_v, par_v, rows_v, ot_v,
        is0, is1, gs0, gs1, ws0, ws1):
    wid = lax.axis_index("s") * NC + lax.axis_index("c")
    iota = lax.iota(jnp.int32, 16)
    isem = (is0, is1)
    gsem = (gs0, gs1)
    wsem = (ws0, ws1)

    def gid_of(i):
        return wid + NW * i

    def start_idx_read(i, gs):
        pltpu.async_copy(idx3_hbm.at[gid_of(i)], raw_v.at[gs], isem[gs])

    def wait_idx_read(gs):
        pltpu.make_async_copy(idx3_hbm.at[0], raw_v.at[gs], isem[gs]).wait()

    def prep(gs):
        for q in range(GB // 16):
            v = raw_v[gs, q // 8, pl.ds(16 * (q % 8), 16)]
            gidx_v[gs, q // 8, pl.ds(16 * (q % 8), 16)] = v >> 1
            par_v[gs, pl.ds(16 * q, 16)] = (v & 1) * 64

    def start_gather(gs, s4, rslot):
        for q in range(2):
            pltpu.async_copy(pair_hbm.at[gidx_v.at[gs, 2 * s4 + q]],
                             rows_v.at[rslot, pl.ds(128 * q, 128)], gsem[rslot])

    def wait_gather(rslot):
        for q in range(2):
            pltpu.make_async_copy(pair_hbm.at[gidx_v.at[0, 0]],
                                  rows_v.at[rslot, pl.ds(0, 128)],
                                  gsem[rslot]).wait()

    def select_transpose(gs, s4, rslot):
        p64s = [par_v[gs, pl.ds(BB * s4 + 16 * m, 16)] for m in range(BB // 16)]
        bvecs = [16 * m + iota for m in range(BB // 16)]

        def dgrp(g, c):
            d0 = g * 8
            for dd in range(8):
                vals = [
                    plsc.load_gather(rows_v.at[rslot],
                                     [bvecs[m], p64s[m] + (d0 + dd)])
                    for m in range(BB // 16)
                ]
                for m in range(BB // 16):
                    ot_v[rslot, d0 + dd, pl.ds(16 * m, 16)] = vals[m]
            return c

        lax.fori_loop(0, D // 8, dgrp, 0)

    def start_out_write(i, s4, rslot):
        g = gid_of(i)
        t = g // NGB
        b0 = pl.multiple_of((g % NGB) * GB + BB * s4, BB)
        pltpu.async_copy(ot_v.at[rslot], o_hbm.at[t, :, pl.ds(b0, BB)],
                         wsem[rslot])

    def wait_out_write(rslot):
        pltpu.make_async_copy(ot_v.at[rslot], o_hbm.at[0, :, pl.ds(0, BB)],
                              wsem[rslot]).wait()

    def process_group(i, gs, first, last):
        wait_idx_read(gs)
        if not last:
            start_idx_read(i + 1, 1 - gs)
        prep(gs)
        start_gather(gs, 0, 0)
        start_gather(gs, 1, 1)
        for s4 in range(4):
            rslot = s4 % 2
            wait_gather(rslot)
            if s4 < 2:
                if not first:
                    wait_out_write(rslot)
                else:
                    pass
            else:
                wait_out_write(rslot)
            select_transpose(gs, s4, rslot)
            start_out_write(i, s4, rslot)
            if s4 + 2 < 4:
                start_gather(gs, s4 + 2, rslot)

    start_idx_read(0, 0)

    def step2(i2, c):
        i = 2 * i2

        @pl.when(i2 == 0)
        def _():
            process_group(i, 0, True, False)
        @pl.when(i2 > 0)
        def _():
            process_group(i, 0, False, False)
        process_group(i + 1, 1, False, False)
        return c

    lax.fori_loop(0, (G_PER_W - 1) // 2, step2, 0)
    process_group(G_PER_W - 1, 0, False, True)

    wait_out_write(0)
    wait_out_write(1)


def kernel(idx, layer_matrix):
    pair = _k1(layer_matrix.T, _tail_pairs(layer_matrix))
    idx3 = idx.T.astype(jnp.int32).reshape(GROUPS, 8, 128)
    o = _k2(idx3, pair)
    return o.transpose(2, 0, 1)


def _tail_pairs(layer_matrix):
    lm_tail = lax.slice(layer_matrix, (NBF * CB, 0), (V, D))
    return jnp.pad(lm_tail, ((0, 2 * TAIL_PAD - TAIL), (0, 0))).reshape(TAIL_PAD, 128)


# speedup vs baseline: 1.4302x; 1.1463x over previous
import functools

import jax
import jax.numpy as jnp
from jax import lax
from jax.experimental import pallas as pl
from jax.experimental.pallas import tpu as pltpu
from jax.experimental.pallas import tpu_sc as plsc

BATCH = 4096
HIST = 200
D = 64
V = 1000002
NC, NS = 2, 16
NW = NC * NS

_mesh = plsc.VectorSubcoreMesh(core_axis_name="c", subcore_axis_name="s")
_tiled = pltpu.CompilerParams(use_tc_tiling_on_sc=True, needs_layout_passes=False)
_linear = pltpu.CompilerParams(use_tc_tiling_on_sc=False, needs_layout_passes=False)

RB = 256
NBF = V // RB
TAIL = V - NBF * RB
TAIL_PAD = 40
K1_ITER = NBF // NW + 1
VPP = NBF * RB // 2 + TAIL_PAD


@functools.partial(
    pl.kernel,
    out_type=jax.ShapeDtypeStruct((VPP, 128), jnp.float32),
    mesh=_mesh,
    scratch_types=[
        pltpu.VMEM((2, RB, D), jnp.float32),
        pltpu.VMEM((2, RB // 2, 128), jnp.float32),
        pltpu.SemaphoreType.DMA,
        pltpu.SemaphoreType.DMA,
        pltpu.SemaphoreType.DMA,
        pltpu.SemaphoreType.DMA,
    ],
    compiler_params=_tiled,
)
def _k1(t_hbm, tail_hbm, pair_hbm, blk_v, out_v, rs0, rs1, ws0, ws1):
    wid = lax.axis_index("s") * NC + lax.axis_index("c")
    rsem = (rs0, rs1)
    wsem = (ws0, ws1)
    nblk = (NBF - wid + NW - 1) // NW

    def r0_of(it):
        return pl.multiple_of((wid + NW * it) * RB, RB)

    def start_read(it, slot):
        pltpu.async_copy(t_hbm.at[pl.ds(r0_of(it), RB)], blk_v.at[slot],
                         rsem[slot])

    def wait_read(slot):
        pltpu.make_async_copy(t_hbm.at[pl.ds(0, RB)], blk_v.at[slot],
                              rsem[slot]).wait()

    def start_write(it, slot):
        pltpu.async_copy(
            out_v.at[slot],
            pair_hbm.at[pl.ds(pl.multiple_of(r0_of(it) // 2, RB // 2), RB // 2)],
            wsem[slot])

    def wait_write(slot):
        pltpu.make_async_copy(out_v.at[slot], pair_hbm.at[pl.ds(0, RB // 2)],
                              wsem[slot]).wait()

    def compact_block(slot):
        def jgrp(g, c):
            j0 = g * 8
            for jj in range(8):
                j = j0 + jj
                for s in range(4):
                    out_v[slot, j, pl.ds(16 * s, 16)] = \
                        blk_v[slot, 2 * j, pl.ds(16 * s, 16)]
                for s in range(4):
                    out_v[slot, j, pl.ds(64 + 16 * s, 16)] = \
                        blk_v[slot, 2 * j + 1, pl.ds(16 * s, 16)]
            return c

        lax.fori_loop(0, RB // 16, jgrp, 0)

    @pl.when(nblk >= 1)
    def _():
        start_read(0, 0)
    @pl.when(nblk >= 2)
    def _():
        start_read(1, 1)

    def step2(it2, c):
        for slot in range(2):
            it = 2 * it2 + slot

            @pl.when(it < nblk)
            def _():
                wait_read(slot)
                @pl.when(it >= 2)
                def _():
                    wait_write(slot)
                compact_block(slot)
                start_write(it, slot)
                @pl.when(it + 2 < nblk)
                def _():
                    start_read(it + 2, slot)
        return c

    lax.fori_loop(0, (K1_ITER + 1) // 2, step2, 0)

    for slot in range(2):
        @pl.when(nblk > slot)
        def _():
            wait_write(slot)

    @pl.when(wid == 0)
    def _():
        pltpu.sync_copy(tail_hbm, out_v.at[0, pl.ds(0, TAIL_PAD)])
        pltpu.sync_copy(out_v.at[0, pl.ds(0, TAIL_PAD)],
                        pair_hbm.at[pl.ds(NBF * RB // 2, TAIL_PAD)])


BB = 256
NBB = BATCH // BB
UNITS = HIST * NBB
K2_ITER = UNITS // NW
SKP = 129


@functools.partial(
    pl.kernel,
    out_type=jax.ShapeDtypeStruct((HIST, D, BATCH), jnp.float32),
    mesh=_mesh,
    scratch_types=[
        pltpu.VMEM((2, BB), jnp.int32),
        pltpu.VMEM((2, 2, 128), jnp.int32),
        pltpu.VMEM((2, BB), jnp.int32),
        pltpu.VMEM((2, BB, 128), jnp.float32),
        pltpu.VMEM((BB * SKP + 16,), jnp.float32),
        pltpu.VMEM((D, BB), jnp.float32),
        pltpu.SemaphoreType.DMA,
        pltpu.SemaphoreType.DMA,
        pltpu.SemaphoreType.DMA,
        pltpu.SemaphoreType.DMA,
        pltpu.SemaphoreType.DMA,
        pltpu.SemaphoreType.DMA,
    ],
    compiler_params=_tiled,
)
def _k2(idxT_hbm, pair_hbm, o_hbm, raw_v, gidx_v, par_v, rows_v, skew_v, ot_v,
        is0, is1, gs0, gs1, ws0, ws1):
    wid = lax.axis_index("s") * NC + lax.axis_index("c")
    iota = lax.iota(jnp.int32, 16)
    isem = (is0, is1)
    gsem = (gs0, gs1)
    wsem = (ws0, ws1)

    def tb_of(it):
        u = wid + NW * it
        return u // NBB, pl.multiple_of((u % NBB) * BB, BB)

    def start_idx_read(it, slot):
        t, b0 = tb_of(it)
        pltpu.async_copy(idxT_hbm.at[t, pl.ds(b0, BB)], raw_v.at[slot],
                         isem[slot])

    def prep_and_gather(slot):
        pltpu.make_async_copy(idxT_hbm.at[0, pl.ds(0, BB)], raw_v.at[slot],
                              isem[slot]).wait()
        for q in range(BB // 16):
            v = raw_v[slot, pl.ds(16 * q, 16)]
            gidx_v[slot, q // 8, pl.ds(16 * (q % 8), 16)] = v >> 1
            par_v[slot, pl.ds(16 * q, 16)] = (v & 1) * 64
        for q in range(BB // 128):
            pltpu.async_copy(pair_hbm.at[gidx_v.at[slot, q]],
                             rows_v.at[slot, pl.ds(128 * q, 128)], gsem[slot])

    def wait_gathers(slot):
        for q in range(BB // 128):
            pltpu.make_async_copy(pair_hbm.at[gidx_v.at[slot, 0]],
                                  rows_v.at[slot, pl.ds(0, 128)],
                                  gsem[slot]).wait()

    def select_transpose(slot):
        cvecs = [iota + 16 * u for u in range(8)]

        def bgrp(g, c):
            b0 = g * 8
            for bb in range(8):
                b = b0 + bb
                ab = b * SKP
                for u in range(8):
                    v = rows_v[slot, b, pl.ds(16 * u, 16)]
                    plsc.store_scatter(skew_v, [cvecs[u] + ab], v)
            return c

        lax.fori_loop(0, BB // 8, bgrp, 0)

        bvecs = [16 * m + iota for m in range(BB // 16)]
        fvecs = [bvecs[m] * SKP + par_v[slot, pl.ds(16 * m, 16)]
                 for m in range(BB // 16)]

        def dgrp(g, c):
            d0 = g * 8
            for dd in range(8):
                d = d0 + dd
                for m in range(BB // 16):
                    vals = plsc.load_gather(skew_v, [fvecs[m] + d])
                    ot_v[d, pl.ds(16 * m, 16)] = vals
            return c

        lax.fori_loop(0, D // 8, dgrp, 0)

    def start_out_write(it):
        t, b0 = tb_of(it)
        pltpu.async_copy(ot_v, o_hbm.at[t, :, pl.ds(b0, BB)], ws0)

    def wait_out_write():
        pltpu.make_async_copy(ot_v, o_hbm.at[0, :, pl.ds(0, BB)], ws0).wait()

    start_idx_read(0, 0)
    prep_and_gather(0)
    start_idx_read(1, 1)

    def step2(it2, c):
        for slot in range(2):
            it = 2 * it2 + slot
            oslot = 1 - slot
            @pl.when(it + 1 < K2_ITER)
            def _():
                prep_and_gather(oslot)
            wait_gathers(slot)
            @pl.when(it >= 1)
            def _():
                wait_out_write()
            select_transpose(slot)
            start_out_write(it)
            @pl.when(it + 2 < K2_ITER)
            def _():
                start_idx_read(it + 2, slot)
        return c

    lax.fori_loop(0, K2_ITER // 2, step2, 0)

    wait_out_write()


def kernel(idx, layer_matrix):
    pair = _k1(layer_matrix, _tail_pairs(layer_matrix))
    idxT = idx.T.astype(jnp.int32)
    o = _k2(idxT, pair)
    return o.transpose(2, 0, 1)


def _tail_pairs(layer_matrix):
    lm_tail = lax.slice(layer_matrix, (NBF * RB, 0), (V, D))
    return jnp.pad(lm_tail, ((0, 2 * TAIL_PAD - TAIL), (0, 0))).reshape(TAIL_PAD, 128)


# speedup vs baseline: 2.6468x; 1.8507x over previous
import functools

import jax
import jax.numpy as jnp
from jax import lax
from jax.experimental import pallas as pl
from jax.experimental.pallas import tpu as pltpu
from jax.experimental.pallas import tpu_sc as plsc

BATCH = 4096
HIST = 200
D = 64
B = BATCH * HIST
NC, NS = 2, 16
NW = NC * NS
RPT = 128
XF = B // (NW * RPT)
NBUF = 4

_mesh = plsc.VectorSubcoreMesh(core_axis_name="c", subcore_axis_name="s")


@functools.partial(
    pl.kernel,
    out_type=jax.ShapeDtypeStruct((B, 2 * D), jnp.float32),
    mesh=_mesh,
    scratch_types=[
        pltpu.VMEM((XF, RPT), jnp.int32),
        pltpu.VMEM((NBUF, RPT, D), jnp.float32),
        pltpu.SemaphoreType.DMA,
        pltpu.SemaphoreType.DMA,
        pltpu.SemaphoreType.DMA,
        pltpu.SemaphoreType.DMA,
        pltpu.SemaphoreType.DMA,
        pltpu.SemaphoreType.DMA,
        pltpu.SemaphoreType.DMA,
        pltpu.SemaphoreType.DMA,
    ],
    compiler_params=pltpu.CompilerParams(use_tc_tiling_on_sc=False),
)
def _gather(idx_hbm, table_hbm, out_hbm, idx_v, rows_v,
            g0, g1, g2, g3, w0, w1, w2, w3):
    wid = lax.axis_index("s") * NC + lax.axis_index("c")
    gsem = (g0, g1, g2, g3)
    wsem = (w0, w1, w2, w3)
    pltpu.sync_copy(idx_hbm.at[wid], idx_v)
    base = wid * (XF * RPT)

    def start_gather(j, slot):
        pltpu.async_copy(table_hbm.at[idx_v.at[j]], rows_v.at[slot], gsem[slot])

    def wait_gather(slot):
        pltpu.make_async_copy(table_hbm.at[idx_v.at[0]], rows_v.at[slot],
                              gsem[slot]).wait()

    def start_write(j, slot):
        pltpu.async_copy(
            rows_v.at[slot],
            out_hbm.at[pl.ds(base + j * RPT, RPT), pl.ds(0, D)],
            wsem[slot])

    def wait_write(slot):
        pltpu.make_async_copy(rows_v.at[slot],
                              out_hbm.at[pl.ds(0, RPT), pl.ds(0, D)],
                              wsem[slot]).wait()

    def step(j, c):
        start_gather(j, 0)
        wait_gather(0)
        start_write(j, 0)
        wait_write(0)
        return c

    lax.fori_loop(0, XF, step, 0)


def kernel(idx, layer_matrix):
    idx32 = idx.reshape(NW, XF, RPT).astype(jnp.int32)
    out = _gather(idx32, layer_matrix)
    return out[:, :D].reshape(BATCH, HIST, D)


# speedup vs baseline: 2.9567x; 1.1171x over previous
import functools

import jax
import jax.numpy as jnp
from jax import lax
from jax.experimental import pallas as pl
from jax.experimental.pallas import tpu as pltpu
from jax.experimental.pallas import tpu_sc as plsc

BATCH = 4096
HIST = 200
D = 64
B = BATCH * HIST
NC, NS = 2, 16
NW = NC * NS
RPT = 128
XF = B // (NW * RPT)
NBUF = 2

_mesh = plsc.VectorSubcoreMesh(core_axis_name="c", subcore_axis_name="s")


@functools.partial(
    pl.kernel,
    out_type=jax.ShapeDtypeStruct((B, 2 * D), jnp.float32),
    mesh=_mesh,
    scratch_types=[
        pltpu.VMEM((XF, RPT), jnp.int32),
        pltpu.VMEM((NBUF, RPT, D), jnp.float32),
        pltpu.SemaphoreType.DMA,
        pltpu.SemaphoreType.DMA,
        pltpu.SemaphoreType.DMA,
        pltpu.SemaphoreType.DMA,
    ],
    compiler_params=pltpu.CompilerParams(use_tc_tiling_on_sc=False),
)
def _gather(idx_hbm, table_hbm, out_hbm, idx_v, rows_v, g0, g1, w0, w1):
    wid = lax.axis_index("s") * NC + lax.axis_index("c")
    gsem = (g0, g1)
    wsem = (w0, w1)
    pltpu.sync_copy(idx_hbm.at[wid], idx_v)
    base = wid * (XF * RPT)

    def start_gather(j, slot):
        pltpu.async_copy(table_hbm.at[idx_v.at[j]], rows_v.at[slot], gsem[slot])

    def wait_gather(slot):
        pltpu.make_async_copy(table_hbm.at[idx_v.at[0]], rows_v.at[slot],
                              gsem[slot]).wait()

    def start_write(j, slot):
        pltpu.async_copy(
            rows_v.at[slot],
            out_hbm.at[pl.ds(base + j * RPT, RPT), pl.ds(0, D)],
            wsem[slot])

    def wait_write(slot):
        pltpu.make_async_copy(rows_v.at[slot],
                              out_hbm.at[pl.ds(0, RPT), pl.ds(0, D)],
                              wsem[slot]).wait()

    start_gather(0, 0)

    def step2(it, c):
        for s in range(2):
            j = 2 * it + s
            o = 1 - s

            @pl.when(j + 1 < XF)
            def _():
                @pl.when(j >= 1)
                def _():
                    wait_write(o)
                start_gather(j + 1, o)
            wait_gather(s)
            start_write(j, s)
        return c

    lax.fori_loop(0, XF // 2, step2, 0)
    wait_write(0)
    wait_write(1)


def kernel(idx, layer_matrix):
    idx32 = idx.reshape(NW, XF, RPT).astype(jnp.int32)
    out = _gather(idx32, layer_matrix)
    return out[:, :D].reshape(BATCH, HIST, D)


# speedup vs baseline: 3.0383x; 1.0276x over previous
import functools

import jax
import jax.numpy as jnp
from jax import lax
from jax.experimental import pallas as pl
from jax.experimental.pallas import tpu as pltpu
from jax.experimental.pallas import tpu_sc as plsc

BATCH = 4096
HIST = 200
D = 64
B = BATCH * HIST
NC, NS = 2, 16
NW = NC * NS
RPT = 128
XF = B // (NW * RPT)
NBUF = 4

_mesh = plsc.VectorSubcoreMesh(core_axis_name="c", subcore_axis_name="s")


@functools.partial(
    pl.kernel,
    out_type=jax.ShapeDtypeStruct((B, 2 * D), jnp.float32),
    mesh=_mesh,
    scratch_types=[
        pltpu.VMEM((XF, RPT), jnp.int32),
        pltpu.VMEM((NBUF, RPT, D), jnp.float32),
        pltpu.SemaphoreType.DMA,
        pltpu.SemaphoreType.DMA,
        pltpu.SemaphoreType.DMA,
        pltpu.SemaphoreType.DMA,
        pltpu.SemaphoreType.DMA,
        pltpu.SemaphoreType.DMA,
        pltpu.SemaphoreType.DMA,
        pltpu.SemaphoreType.DMA,
    ],
    compiler_params=pltpu.CompilerParams(use_tc_tiling_on_sc=False),
)
def _gather(idx_hbm, table_hbm, out_hbm, idx_v, rows_v,
            g0, g1, g2, g3, w0, w1, w2, w3):
    wid = lax.axis_index("s") * NC + lax.axis_index("c")
    gsem = (g0, g1, g2, g3)
    wsem = (w0, w1, w2, w3)
    pltpu.sync_copy(idx_hbm.at[wid], idx_v)
    base = wid * (XF * RPT)

    def start_gather(j, slot):
        pltpu.async_copy(table_hbm.at[idx_v.at[j]], rows_v.at[slot], gsem[slot])

    def wait_gather(slot):
        pltpu.make_async_copy(table_hbm.at[idx_v.at[0]], rows_v.at[slot],
                              gsem[slot]).wait()

    def start_write(j, slot):
        pltpu.async_copy(
            rows_v.at[slot],
            out_hbm.at[pl.ds(base + j * RPT, RPT), pl.ds(0, D)],
            wsem[slot])

    def wait_write(slot):
        pltpu.make_async_copy(rows_v.at[slot],
                              out_hbm.at[pl.ds(0, RPT), pl.ds(0, D)],
                              wsem[slot]).wait()

    for s in range(NBUF - 1):
        start_gather(s, s)

    def step4(it, c):
        for s in range(NBUF):
            j = NBUF * it + s
            o = (s + NBUF - 1) % NBUF

            @pl.when(j + NBUF - 1 < XF)
            def _():
                @pl.when(j >= 1)
                def _():
                    wait_write(o)
                start_gather(j + NBUF - 1, o)
            wait_gather(s)
            start_write(j, s)
        return c

    lax.fori_loop(0, XF // NBUF, step4, 0)
    for s in range(NBUF):
        wait_write(s)


def kernel(idx, layer_matrix):
    idx32 = idx.reshape(NW, XF, RPT).astype(jnp.int32)
    out = _gather(idx32, layer_matrix)
    return out[:, :D].reshape(BATCH, HIST, D)
